# baseline (speedup 1.0000x reference)
import functools

import jax
import jax.numpy as jnp
from jax import lax
from jax.experimental import pallas as pl
from jax.experimental.pallas import tpu as pltpu
from jax.experimental.pallas import tpu_sc as plsc

_NC = 2
_NS = 16
_LANES = 128
_DUMMY = 32
_NBUF = 2
_NPHASE = 2


def _sc_aggregate(x, src2d, dst2d):
  N, D = x.shape
  G = src2d.shape[0]
  NW = _NC * _NS
  gpw = G // NW
  gps = gpw // _NPHASE
  rpt = (N // (8 * _NS)) * 8
  tail = N - rpt * _NS

  mesh = plsc.VectorSubcoreMesh(core_axis_name="c", subcore_axis_name="s")

  @functools.partial(
      pl.kernel,
      out_type=(jax.ShapeDtypeStruct((N, D), jnp.float32),
                jax.ShapeDtypeStruct((N, D), jnp.float32)),
      mesh=mesh,
      scratch_types=[
          pltpu.VMEM_SHARED((N + _DUMMY, D), jnp.float32),
          pltpu.VMEM((gps, _LANES), jnp.int32),
          pltpu.VMEM((gps, _LANES), jnp.int32),
          [pltpu.VMEM((_LANES, D), jnp.float32) for _ in range(_NBUF)],
          pltpu.SemaphoreType.DMA,
          [pltpu.SemaphoreType.DMA for _ in range(_NBUF)],
          [pltpu.SemaphoreType.DMA for _ in range(_NBUF)],
      ],
  )
  def agg_kernel(x_hbm, src_hbm, dst_hbm, out0_hbm, out1_hbm,
                 agg_sh, srcv, dstv, rows, isem, gsem, ssem):
    c = lax.axis_index("c")
    s = lax.axis_index("s")
    wid = s * _NC + c

    base_g = wid * gpw
    pltpu.async_copy(src_hbm.at[pl.ds(base_g, gps)], srcv, isem)
    pltpu.async_copy(dst_hbm.at[pl.ds(base_g, gps)], dstv, isem)

    zeros16 = jnp.zeros((16,), jnp.float32)

    def zrow(i, carry):
      for j in range(D // 16):
        rows[0][i, pl.ds(j * 16, 16)] = zeros16
      return carry

    lax.fori_loop(0, _LANES, zrow, 0)

    def zero_span(base_row, nrows):
      off = 0
      while off < nrows:
        sz = min(_LANES, nrows - off)
        pltpu.sync_copy(rows[0].at[pl.ds(0, sz)],
                        agg_sh.at[pl.ds(base_row + off, sz)])
        off += sz

    base_row = s * rpt
    zero_span(base_row, rpt)
    if tail:
      @pl.when(s == 0)
      def _():
        zero_span(_NS * rpt, tail)

    pltpu.make_async_copy(src_hbm.at[pl.ds(base_g, gps)], srcv, isem).wait()
    pltpu.make_async_copy(dst_hbm.at[pl.ds(base_g, gps)], dstv, isem).wait()
    plsc.subcore_barrier()

    def gather(g, b):
      pltpu.async_copy(x_hbm.at[srcv.at[g]], rows[b], gsem[b])

    def gather_wait(g, b):
      pltpu.make_async_copy(x_hbm.at[srcv.at[g]], rows[b], gsem[b]).wait()

    def scatter(g, b):
      pltpu.async_copy(rows[b], agg_sh.at[dstv.at[g]], ssem[b], add=True)

    def scatter_wait(g, b):
      pltpu.make_async_copy(rows[b], agg_sh.at[dstv.at[g]], ssem[b]).wait()

    def pair(o, carry):
      g0 = o * 2
      g1 = g0 + 1
      gather_wait(g0, 0)
      scatter(g0, 0)

      @pl.when(o > 0)
      def _():
        scatter_wait(g1 - 2, 1)

      gather(g1, 1)
      gather_wait(g1, 1)
      scatter(g1, 1)

      @pl.when(g0 + 2 < gps)
      def _():
        scatter_wait(g0, 0)
        gather(g0 + 2, 0)

      return carry

    for p in range(_NPHASE):
      if p > 0:
        pltpu.sync_copy(src_hbm.at[pl.ds(base_g + p * gps, gps)], srcv)
        pltpu.sync_copy(dst_hbm.at[pl.ds(base_g + p * gps, gps)], dstv)
      gather(0, 0)
      lax.fori_loop(0, gps // 2, pair, 0, unroll=False)
      scatter_wait(gps - 2, 0)
      scatter_wait(gps - 1, 1)
    plsc.subcore_barrier()

    def copy_out(out_hbm):
      pltpu.sync_copy(agg_sh.at[pl.ds(base_row, rpt)],
                      out_hbm.at[pl.ds(base_row, rpt)])
      if tail:
        @pl.when(s == 0)
        def _():
          pltpu.sync_copy(agg_sh.at[pl.ds(_NS * rpt, tail)],
                          out_hbm.at[pl.ds(_NS * rpt, tail)])

    @pl.when(c == 0)
    def _():
      copy_out(out0_hbm)

    @pl.when(c == 1)
    def _():
      copy_out(out1_hbm)

  return agg_kernel(x, src2d, dst2d)


def _tc_mlp(x, a0, a1, scale, W1f, c1, W2f, c2):
  N, D = x.shape
  H = W1f.shape[1]
  BN = 1000
  grid = (N // BN,)

  def body(scale_ref, x_ref, a0_ref, a1_ref, w1_ref, c1_ref, w2_ref, c2_ref,
           o_ref):
    h = scale_ref[0, 0] * x_ref[...] + a0_ref[...] + a1_ref[...]
    y = jnp.dot(h, w1_ref[...], preferred_element_type=jnp.float32)
    y = jnp.maximum(y + c1_ref[...], 0.0)
    y = jnp.dot(y, w2_ref[...], preferred_element_type=jnp.float32)
    o_ref[...] = jnp.maximum(y + c2_ref[...], 0.0)

  return pl.pallas_call(
      body,
      grid=grid,
      in_specs=[
          pl.BlockSpec(memory_space=pltpu.SMEM),
          pl.BlockSpec((BN, D), lambda i: (i, 0)),
          pl.BlockSpec((BN, D), lambda i: (i, 0)),
          pl.BlockSpec((BN, D), lambda i: (i, 0)),
          pl.BlockSpec((D, H), lambda i: (0, 0)),
          pl.BlockSpec((1, H), lambda i: (0, 0)),
          pl.BlockSpec((H, D), lambda i: (0, 0)),
          pl.BlockSpec((1, D), lambda i: (0, 0)),
      ],
      out_specs=pl.BlockSpec((BN, D), lambda i: (i, 0)),
      out_shape=jax.ShapeDtypeStruct((N, D), jnp.float32),
  )(scale, x, a0, a1, W1f, c1, W2f, c2)


def kernel(x, ei, eps, W1, b1, g1, beta1, W2, b2, g2, beta2):
  N, D = x.shape
  E = ei.shape[1]

  unit = _LANES * _NC * _NS * 8
  E_pad = -(-E // unit) * unit
  pad = E_pad - E
  src = ei[0]
  dst = ei[1]
  if pad:
    fill = jnp.arange(pad, dtype=jnp.int32)
    src = jnp.concatenate([src, fill % N])
    dst = jnp.concatenate([dst, N + (fill % _DUMMY)])
  src2d = src.reshape(E_pad // _LANES, _LANES)
  dst2d = dst.reshape(E_pad // _LANES, _LANES)

  agg0, agg1 = _sc_aggregate(x, src2d, dst2d)

  bn = 1.0 / jnp.sqrt(1.0 + 1e-5)
  s1 = bn * g1
  W1f = W1 * s1[None, :]
  c1 = (b1 * s1 + beta1)[None, :]
  s2 = bn * g2
  W2f = W2 * s2[None, :]
  c2 = (b2 * s2 + beta2)[None, :]
  scale = jnp.reshape(1.0 + eps, (1, 1))

  return _tc_mlp(x, agg0, agg1, scale, W1f, c1, W2f, c2)


# speedup vs baseline: 1.0035x; 1.0035x over previous
import functools

import jax
import jax.numpy as jnp
from jax import lax
from jax.experimental import pallas as pl
from jax.experimental.pallas import tpu as pltpu
from jax.experimental.pallas import tpu_sc as plsc

_NC = 2
_NS = 16
_LANES = 128
_DUMMY = 32
_NBUF = 2
_NPHASE = 2


def _sc_aggregate(x, src2d, dst2d):
  N, D = x.shape
  G = src2d.shape[0]
  NW = _NC * _NS
  gpw = G // NW
  gps = gpw // _NPHASE
  rpt = (N // (8 * _NS)) * 8
  tail = N - rpt * _NS

  mesh = plsc.VectorSubcoreMesh(core_axis_name="c", subcore_axis_name="s")

  @functools.partial(
      pl.kernel,
      out_type=(jax.ShapeDtypeStruct((N, D), jnp.float32),
                jax.ShapeDtypeStruct((N, D), jnp.float32)),
      mesh=mesh,
      scratch_types=[
          pltpu.VMEM_SHARED((N + _DUMMY, D), jnp.float32),
          pltpu.VMEM((gps, _LANES), jnp.int32),
          pltpu.VMEM((gps, _LANES), jnp.int32),
          [pltpu.VMEM((_LANES, D), jnp.float32) for _ in range(_NBUF)],
          pltpu.SemaphoreType.DMA,
          [pltpu.SemaphoreType.DMA for _ in range(_NBUF)],
          [pltpu.SemaphoreType.DMA for _ in range(_NBUF)],
      ],
  )
  def agg_kernel(x_hbm, src_hbm, dst_hbm, out0_hbm, out1_hbm,
                 agg_sh, srcv, dstv, rows, isem, gsem, ssem):
    c = lax.axis_index("c")
    s = lax.axis_index("s")
    wid = s * _NC + c

    base_g = wid * gpw
    pltpu.async_copy(src_hbm.at[pl.ds(base_g, gps)], srcv, isem)
    pltpu.async_copy(dst_hbm.at[pl.ds(base_g, gps)], dstv, isem)

    zeros16 = jnp.zeros((16,), jnp.float32)

    def zrow(i, carry):
      for j in range(D // 16):
        rows[0][i, pl.ds(j * 16, 16)] = zeros16
      return carry

    lax.fori_loop(0, _LANES, zrow, 0)

    def zero_span(base_row, nrows):
      off = 0
      while off < nrows:
        sz = min(_LANES, nrows - off)
        pltpu.sync_copy(rows[0].at[pl.ds(0, sz)],
                        agg_sh.at[pl.ds(base_row + off, sz)])
        off += sz

    base_row = s * rpt
    zero_span(base_row, rpt)
    if tail:
      @pl.when(s == 0)
      def _():
        zero_span(_NS * rpt, tail)

    pltpu.make_async_copy(src_hbm.at[pl.ds(base_g, gps)], srcv, isem).wait()
    pltpu.make_async_copy(dst_hbm.at[pl.ds(base_g, gps)], dstv, isem).wait()
    plsc.subcore_barrier()

    def gather(g, b):
      pltpu.async_copy(x_hbm.at[srcv.at[g]], rows[b], gsem[b])

    def gather_wait(g, b):
      pltpu.make_async_copy(x_hbm.at[srcv.at[g]], rows[b], gsem[b]).wait()

    def scatter(g, b):
      pltpu.async_copy(rows[b], agg_sh.at[dstv.at[g]], ssem[b], add=True)

    def scatter_wait(g, b):
      pltpu.make_async_copy(rows[b], agg_sh.at[dstv.at[g]], ssem[b]).wait()

    def pair(o, carry):
      g0 = o * 2
      g1 = g0 + 1
      gather_wait(g0, 0)
      scatter(g0, 0)

      @pl.when(o > 0)
      def _():
        scatter_wait(g1 - 2, 1)

      gather(g1, 1)
      gather_wait(g1, 1)
      scatter(g1, 1)

      @pl.when(g0 + 2 < gps)
      def _():
        scatter_wait(g0, 0)
        gather(g0 + 2, 0)

      return carry

    for p in range(_NPHASE):
      if p > 0:
        pltpu.sync_copy(src_hbm.at[pl.ds(base_g + p * gps, gps)], srcv)
        pltpu.sync_copy(dst_hbm.at[pl.ds(base_g + p * gps, gps)], dstv)
      gather(0, 0)
      lax.fori_loop(0, gps // 2, pair, 0, unroll=2)
      scatter_wait(gps - 2, 0)
      scatter_wait(gps - 1, 1)
    plsc.subcore_barrier()

    def copy_out(out_hbm):
      pltpu.sync_copy(agg_sh.at[pl.ds(base_row, rpt)],
                      out_hbm.at[pl.ds(base_row, rpt)])
      if tail:
        @pl.when(s == 0)
        def _():
          pltpu.sync_copy(agg_sh.at[pl.ds(_NS * rpt, tail)],
                          out_hbm.at[pl.ds(_NS * rpt, tail)])

    @pl.when(c == 0)
    def _():
      copy_out(out0_hbm)

    @pl.when(c == 1)
    def _():
      copy_out(out1_hbm)

  return agg_kernel(x, src2d, dst2d)


def _tc_mlp(x, a0, a1, scale, W1f, c1, W2f, c2):
  N, D = x.shape
  H = W1f.shape[1]
  BN = 1000
  grid = (N // BN,)

  def body(scale_ref, x_ref, a0_ref, a1_ref, w1_ref, c1_ref, w2_ref, c2_ref,
           o_ref):
    h = scale_ref[0, 0] * x_ref[...] + a0_ref[...] + a1_ref[...]
    y = jnp.dot(h, w1_ref[...], preferred_element_type=jnp.float32)
    y = jnp.maximum(y + c1_ref[...], 0.0)
    y = jnp.dot(y, w2_ref[...], preferred_element_type=jnp.float32)
    o_ref[...] = jnp.maximum(y + c2_ref[...], 0.0)

  return pl.pallas_call(
      body,
      grid=grid,
      in_specs=[
          pl.BlockSpec(memory_space=pltpu.SMEM),
          pl.BlockSpec((BN, D), lambda i: (i, 0)),
          pl.BlockSpec((BN, D), lambda i: (i, 0)),
          pl.BlockSpec((BN, D), lambda i: (i, 0)),
          pl.BlockSpec((D, H), lambda i: (0, 0)),
          pl.BlockSpec((1, H), lambda i: (0, 0)),
          pl.BlockSpec((H, D), lambda i: (0, 0)),
          pl.BlockSpec((1, D), lambda i: (0, 0)),
      ],
      out_specs=pl.BlockSpec((BN, D), lambda i: (i, 0)),
      out_shape=jax.ShapeDtypeStruct((N, D), jnp.float32),
  )(scale, x, a0, a1, W1f, c1, W2f, c2)


def kernel(x, ei, eps, W1, b1, g1, beta1, W2, b2, g2, beta2):
  N, D = x.shape
  E = ei.shape[1]

  unit = _LANES * _NC * _NS * 8
  E_pad = -(-E // unit) * unit
  pad = E_pad - E
  src = ei[0]
  dst = ei[1]
  if pad:
    fill = jnp.arange(pad, dtype=jnp.int32)
    src = jnp.concatenate([src, fill % N])
    dst = jnp.concatenate([dst, N + (fill % _DUMMY)])
  src2d = src.reshape(E_pad // _LANES, _LANES)
  dst2d = dst.reshape(E_pad // _LANES, _LANES)

  agg0, agg1 = _sc_aggregate(x, src2d, dst2d)

  bn = 1.0 / jnp.sqrt(1.0 + 1e-5)
  s1 = bn * g1
  W1f = W1 * s1[None, :]
  c1 = (b1 * s1 + beta1)[None, :]
  s2 = bn * g2
  W2f = W2 * s2[None, :]
  c2 = (b2 * s2 + beta2)[None, :]
  scale = jnp.reshape(1.0 + eps, (1, 1))

  return _tc_mlp(x, agg0, agg1, scale, W1f, c1, W2f, c2)


# speedup vs baseline: 1.0051x; 1.0016x over previous
import functools

import jax
import jax.numpy as jnp
from jax import lax
from jax.experimental import pallas as pl
from jax.experimental.pallas import tpu as pltpu
from jax.experimental.pallas import tpu_sc as plsc

_NC = 2
_NS = 16
_LANES = 128
_DUMMY = 32
_NBUF = 2
_NPHASE = 2


def _sc_aggregate(x, src1, dst1):
  N, D = x.shape
  G = src1.shape[0] // _LANES
  NW = _NC * _NS
  gpw = G // NW
  gps = gpw // _NPHASE
  rpt = (N // (8 * _NS)) * 8
  tail = N - rpt * _NS

  mesh = plsc.VectorSubcoreMesh(core_axis_name="c", subcore_axis_name="s")

  @functools.partial(
      pl.kernel,
      out_type=(jax.ShapeDtypeStruct((N, D), jnp.float32),
                jax.ShapeDtypeStruct((N, D), jnp.float32)),
      mesh=mesh,
      scratch_types=[
          pltpu.VMEM_SHARED((N + _DUMMY, D), jnp.float32),
          pltpu.VMEM((gps * _LANES,), jnp.int32),
          pltpu.VMEM((gps * _LANES,), jnp.int32),
          [pltpu.VMEM((_LANES, D), jnp.float32) for _ in range(_NBUF)],
          pltpu.SemaphoreType.DMA,
          [pltpu.SemaphoreType.DMA for _ in range(_NBUF)],
          [pltpu.SemaphoreType.DMA for _ in range(_NBUF)],
      ],
  )
  def agg_kernel(x_hbm, src_hbm, dst_hbm, out0_hbm, out1_hbm,
                 agg_sh, srcv, dstv, rows, isem, gsem, ssem):
    c = lax.axis_index("c")
    s = lax.axis_index("s")
    wid = s * _NC + c

    base_e = wid * gpw * _LANES
    pltpu.async_copy(src_hbm.at[pl.ds(base_e, gps * _LANES)], srcv, isem)
    pltpu.async_copy(dst_hbm.at[pl.ds(base_e, gps * _LANES)], dstv, isem)

    zeros16 = jnp.zeros((16,), jnp.float32)

    def zrow(i, carry):
      for j in range(D // 16):
        rows[0][i, pl.ds(j * 16, 16)] = zeros16
      return carry

    lax.fori_loop(0, _LANES, zrow, 0)

    def zero_span(base_row, nrows):
      off = 0
      while off < nrows:
        sz = min(_LANES, nrows - off)
        pltpu.sync_copy(rows[0].at[pl.ds(0, sz)],
                        agg_sh.at[pl.ds(base_row + off, sz)])
        off += sz

    base_row = s * rpt
    zero_span(base_row, rpt)
    if tail:
      @pl.when(s == 0)
      def _():
        zero_span(_NS * rpt, tail)

    pltpu.make_async_copy(src_hbm.at[pl.ds(base_e, gps * _LANES)], srcv,
                          isem).wait()
    pltpu.make_async_copy(dst_hbm.at[pl.ds(base_e, gps * _LANES)], dstv,
                          isem).wait()
    plsc.subcore_barrier()

    def gidx(ref, g):
      return ref.at[pl.ds(pl.multiple_of(g * _LANES, _LANES), _LANES)]

    def gather(g, b):
      pltpu.async_copy(x_hbm.at[gidx(srcv, g)], rows[b], gsem[b])

    def gather_wait(g, b):
      pltpu.make_async_copy(x_hbm.at[gidx(srcv, g)], rows[b], gsem[b]).wait()

    def scatter(g, b):
      pltpu.async_copy(rows[b], agg_sh.at[gidx(dstv, g)], ssem[b], add=True)

    def scatter_wait(g, b):
      pltpu.make_async_copy(rows[b], agg_sh.at[gidx(dstv, g)], ssem[b]).wait()

    def pair(o, carry):
      g0 = o * 2
      g1 = g0 + 1
      gather_wait(g0, 0)
      scatter(g0, 0)

      @pl.when(o > 0)
      def _():
        scatter_wait(g1 - 2, 1)

      gather(g1, 1)
      gather_wait(g1, 1)
      scatter(g1, 1)

      @pl.when(g0 + 2 < gps)
      def _():
        scatter_wait(g0, 0)
        gather(g0 + 2, 0)

      return carry

    for p in range(_NPHASE):
      if p > 0:
        pltpu.sync_copy(
            src_hbm.at[pl.ds(base_e + p * gps * _LANES, gps * _LANES)], srcv)
        pltpu.sync_copy(
            dst_hbm.at[pl.ds(base_e + p * gps * _LANES, gps * _LANES)], dstv)
      gather(0, 0)
      lax.fori_loop(0, gps // 2, pair, 0, unroll=2)
      scatter_wait(gps - 2, 0)
      scatter_wait(gps - 1, 1)
    plsc.subcore_barrier()

    def copy_out(out_hbm):
      pltpu.sync_copy(agg_sh.at[pl.ds(base_row, rpt)],
                      out_hbm.at[pl.ds(base_row, rpt)])
      if tail:
        @pl.when(s == 0)
        def _():
          pltpu.sync_copy(agg_sh.at[pl.ds(_NS * rpt, tail)],
                          out_hbm.at[pl.ds(_NS * rpt, tail)])

    @pl.when(c == 0)
    def _():
      copy_out(out0_hbm)

    @pl.when(c == 1)
    def _():
      copy_out(out1_hbm)

  return agg_kernel(x, src1, dst1)


def _tc_mlp(x, a0, a1, scale, W1, b1, g1, beta1, W2, b2, g2, beta2):
  N, D = x.shape
  H = W1.shape[1]
  BN = 1000
  grid = (N // BN,)
  bn = float(1.0 / (1.0 + 1e-5) ** 0.5)

  def body(scale_ref, x_ref, a0_ref, a1_ref, w1_ref, b1_ref, g1_ref, bb1_ref,
           w2_ref, b2_ref, g2_ref, bb2_ref, o_ref):
    s1 = bn * g1_ref[...]
    c1 = b1_ref[...] * s1 + bb1_ref[...]
    s2 = bn * g2_ref[...]
    c2 = b2_ref[...] * s2 + bb2_ref[...]
    h = scale_ref[0, 0] * x_ref[...] + a0_ref[...] + a1_ref[...]
    y = jnp.dot(h, w1_ref[...] * s1, preferred_element_type=jnp.float32)
    y = jnp.maximum(y + c1, 0.0)
    y = jnp.dot(y, w2_ref[...] * s2, preferred_element_type=jnp.float32)
    o_ref[...] = jnp.maximum(y + c2, 0.0)

  full = lambda shape: pl.BlockSpec(shape, lambda i: tuple(0 for _ in shape))
  return pl.pallas_call(
      body,
      grid=grid,
      in_specs=[
          pl.BlockSpec(memory_space=pltpu.SMEM),
          pl.BlockSpec((BN, D), lambda i: (i, 0)),
          pl.BlockSpec((BN, D), lambda i: (i, 0)),
          pl.BlockSpec((BN, D), lambda i: (i, 0)),
          full((D, H)), full((1, H)), full((1, H)), full((1, H)),
          full((H, D)), full((1, D)), full((1, D)), full((1, D)),
      ],
      out_specs=pl.BlockSpec((BN, D), lambda i: (i, 0)),
      out_shape=jax.ShapeDtypeStruct((N, D), jnp.float32),
  )(scale, x, a0, a1, W1, b1.reshape(1, H), g1.reshape(1, H),
    beta1.reshape(1, H), W2, b2.reshape(1, D), g2.reshape(1, D),
    beta2.reshape(1, D))


def kernel(x, ei, eps, W1, b1, g1, beta1, W2, b2, g2, beta2):
  N, D = x.shape
  E = ei.shape[1]

  unit = _LANES * _NC * _NS * 8
  E_pad = -(-E // unit) * unit
  pad = E_pad - E
  src = ei[0]
  dst = ei[1]
  if pad:
    fill = jnp.arange(pad, dtype=jnp.int32)
    src = jnp.concatenate([src, fill % N])
    dst = jnp.concatenate([dst, N + (fill % _DUMMY)])
  agg0, agg1 = _sc_aggregate(x, src, dst)

  scale = jnp.reshape(1.0 + eps, (1, 1))
  return _tc_mlp(x, agg0, agg1, scale, W1, b1, g1, beta1, W2, b2, g2, beta2)
